# baseline (speedup 1.0000x reference)
import jax
import jax.numpy as jnp
from jax.experimental import pallas as pl
from jax.experimental.pallas import tpu as pltpu

N = 4096
D = 512
BM = 512


def _body(adj_ref, x_ref, w4_ref, w8_ref, w16_ref, w32_ref, o_ref,
          x16_ref, w16s_ref):
    i = pl.program_id(0)

    @pl.when(i == 0)
    def _init():
        x16_ref[...] = x_ref[...].astype(jnp.bfloat16)
        w16s_ref[0] = w4_ref[...].astype(jnp.bfloat16)
        w16s_ref[1] = w8_ref[...].astype(jnp.bfloat16)
        w16s_ref[2] = w16_ref[...].astype(jnp.bfloat16)
        w16s_ref[3] = w32_ref[...].astype(jnp.bfloat16)

    a16 = adj_ref[...].astype(jnp.bfloat16)
    y = jnp.dot(a16, x16_ref[...], preferred_element_type=jnp.float32)
    y16 = y.astype(jnp.bfloat16)

    def m(k):
        return jnp.maximum(
            jnp.dot(y16, w16s_ref[k], preferred_element_type=jnp.float32), 0.0)

    o_ref[...] = m(0) + m(1) + m(2) + m(3)


@jax.jit
def _run(x, adj, W4, W8, W16, W32):
    grid = (N // BM,)
    w_spec = pl.BlockSpec((D, D), lambda i: (0, 0))
    return pl.pallas_call(
        _body,
        grid=grid,
        in_specs=[
            pl.BlockSpec((BM, N), lambda i: (i, 0)),
            pl.BlockSpec((N, D), lambda i: (0, 0)),
            w_spec, w_spec, w_spec, w_spec,
        ],
        out_specs=pl.BlockSpec((BM, D), lambda i: (i, 0)),
        out_shape=jax.ShapeDtypeStruct((N, D), jnp.float32),
        scratch_shapes=[
            pltpu.VMEM((N, D), jnp.bfloat16),
            pltpu.VMEM((4, D, D), jnp.bfloat16),
        ],
    )(adj, x, W4, W8, W16, W32)


def kernel(x, adj, now_epoch, W4, W8, W16, W32):
    return _run(x, adj, W4, W8, W16, W32)


# speedup vs baseline: 1.0087x; 1.0087x over previous
import jax
import jax.numpy as jnp
from jax.experimental import pallas as pl
from jax.experimental.pallas import tpu as pltpu

N = 4096
D = 512
BM = 512
H = N // 2


def _body(al_ref, ar_ref, x_ref, w4_ref, w8_ref, w16_ref, w32_ref, o_ref):
    xl = x_ref[0:H, :].astype(jnp.bfloat16)
    xr = x_ref[H:N, :].astype(jnp.bfloat16)
    y = jnp.dot(al_ref[...].astype(jnp.bfloat16), xl,
                preferred_element_type=jnp.float32)
    y = y + jnp.dot(ar_ref[...].astype(jnp.bfloat16), xr,
                    preferred_element_type=jnp.float32)
    y16 = y.astype(jnp.bfloat16)

    def m(w_ref):
        w16 = w_ref[...].astype(jnp.bfloat16)
        return jnp.maximum(jnp.dot(y16, w16, preferred_element_type=jnp.float32), 0.0)

    o_ref[...] = m(w4_ref) + m(w8_ref) + m(w16_ref) + m(w32_ref)


@jax.jit
def _run(x, adj, W4, W8, W16, W32):
    grid = (N // BM,)
    w_spec = pl.BlockSpec((D, D), lambda i: (0, 0))
    return pl.pallas_call(
        _body,
        grid=grid,
        in_specs=[
            pl.BlockSpec((BM, H), lambda i: (i, 0)),
            pl.BlockSpec((BM, H), lambda i: (i, 1)),
            pl.BlockSpec((N, D), lambda i: (0, 0)),
            w_spec, w_spec, w_spec, w_spec,
        ],
        out_specs=pl.BlockSpec((BM, D), lambda i: (i, 0)),
        out_shape=jax.ShapeDtypeStruct((N, D), jnp.float32),
    )(adj, adj, x, W4, W8, W16, W32)


def kernel(x, adj, now_epoch, W4, W8, W16, W32):
    return _run(x, adj, W4, W8, W16, W32)


# speedup vs baseline: 1.0155x; 1.0067x over previous
import jax
import jax.numpy as jnp
from jax.experimental import pallas as pl

N = 4096
D = 512
BM = 512


def _body(adj_ref, x_ref, w4_ref, w8_ref, w16_ref, w32_ref, o_ref):
    a16 = adj_ref[...].astype(jnp.bfloat16)
    x16 = x_ref[...].astype(jnp.bfloat16)
    y = jnp.dot(a16, x16, preferred_element_type=jnp.float32)
    y16 = y.astype(jnp.bfloat16)

    def m(w_ref):
        w16 = w_ref[...].astype(jnp.bfloat16)
        return jnp.maximum(jnp.dot(y16, w16, preferred_element_type=jnp.float32), 0.0)

    o_ref[...] = m(w4_ref) + m(w8_ref) + m(w16_ref) + m(w32_ref)


@jax.jit
def _run(x, adj, W4, W8, W16, W32):
    grid = (N // BM,)
    w_spec = pl.BlockSpec((D, D), lambda i: (0, 0))
    return pl.pallas_call(
        _body,
        grid=grid,
        in_specs=[
            pl.BlockSpec((BM, N), lambda i: (i, 0)),
            pl.BlockSpec((N, D), lambda i: (0, 0)),
            w_spec, w_spec, w_spec, w_spec,
        ],
        out_specs=pl.BlockSpec((BM, D), lambda i: (i, 0)),
        out_shape=jax.ShapeDtypeStruct((N, D), jnp.float32),
    )(adj, x, W4, W8, W16, W32)


def kernel(x, adj, now_epoch, W4, W8, W16, W32):
    return _run(x, adj, W4, W8, W16, W32)
